# Initial kernel scaffold; baseline (speedup 1.0000x reference)
import functools

import jax
import jax.numpy as jnp
from jax import lax
from jax.experimental import pallas as pl
from jax.experimental.pallas import tpu as pltpu
from jax.experimental.pallas import tpu_sc as plsc

N = 10000
D = 128
E = 320000
EPS = 1e-5

NC = 2
NS = 16
NW = NC * NS
C = 128
EPT = -(-E // NW)
K = -(-EPT // C)
EPAD = NW * K * C
ROWS_ACC = 10016
RPS = ROWS_ACC // NS

_mesh = plsc.VectorSubcoreMesh(core_axis_name="c", subcore_axis_name="s")


@functools.partial(
    pl.kernel,
    mesh=_mesh,
    out_type=jax.ShapeDtypeStruct((NC, ROWS_ACC, D), jnp.float32),
    scratch_types=[
        pltpu.VMEM((K, C), jnp.int32),
        pltpu.VMEM((K, C), jnp.int32),
        pltpu.VMEM((C, D), jnp.float32),
        pltpu.VMEM_SHARED((ROWS_ACC, D), jnp.float32),
        pltpu.SemaphoreType.DMA,
    ],
)
def _sc_scatter(h_hbm, src_hbm, dst_hbm, zeros_hbm, out_hbm,
                src_v, dst_v, buf, acc, sem):
    c = lax.axis_index("c")
    s = lax.axis_index("s")
    w = s * NC + c
    pltpu.sync_copy(zeros_hbm, acc.at[pl.ds(s * RPS, RPS)])
    pltpu.sync_copy(src_hbm.at[w], src_v)
    pltpu.sync_copy(dst_hbm.at[w], dst_v)
    plsc.subcore_barrier()

    def body(j, carry):
        pltpu.async_copy(h_hbm.at[src_v.at[j]], buf, sem).wait()
        pltpu.sync_copy(buf, acc.at[dst_v.at[j]], add=True)
        return carry

    lax.fori_loop(0, K, body, 0, unroll=False)
    plsc.subcore_barrier()
    pltpu.sync_copy(acc.at[pl.ds(s * RPS, RPS)],
                    out_hbm.at[c, pl.ds(s * RPS, RPS)])


def _tc_mm_bn(p_ref, w_ref, g_ref, b_ref, o_ref):
    agg = p_ref[0, :N, :] + p_ref[1, :N, :]
    z = lax.dot_general(agg, w_ref[...], (((1,), (1,)), ((), ())),
                        preferred_element_type=jnp.float32)
    mean = jnp.mean(z, axis=0, keepdims=True)
    zc = z - mean
    var = jnp.mean(zc * zc, axis=0, keepdims=True)
    o_ref[...] = zc * lax.rsqrt(var + EPS) * g_ref[...] + b_ref[...]


def kernel(h, edge_index, W, gamma, beta):
    src = edge_index[0].astype(jnp.int32)
    dst = edge_index[1].astype(jnp.int32)
    pad = EPAD - E
    src = jnp.concatenate([src, jnp.zeros((pad,), jnp.int32)]).reshape(NW, K, C)
    dst = jnp.concatenate([dst, jnp.full((pad,), N, jnp.int32)]).reshape(NW, K, C)
    zeros = jnp.zeros((RPS, D), jnp.float32)

    partial = _sc_scatter(h, src, dst, zeros)

    out = pl.pallas_call(
        _tc_mm_bn,
        out_shape=jax.ShapeDtypeStruct((N, D), jnp.float32),
    )(partial, W, gamma.reshape(1, D), beta.reshape(1, D))
    return out


# speedup vs baseline: 3.2197x; 3.2197x over previous
import functools

import jax
import jax.numpy as jnp
from jax import lax
from jax.experimental import pallas as pl
from jax.experimental.pallas import tpu as pltpu
from jax.experimental.pallas import tpu_sc as plsc

N = 10000
D = 128
E = 320000
EPS = 1e-5

NC = 2
NS = 16
NW = NC * NS
C = 128
EPT = -(-E // NW)
K = 80
KH = K // 2
KH2 = KH // 2
EPAD = NW * K * C
ROWS_ACC = 10112
RPS = ROWS_ACC // NS

_mesh = plsc.VectorSubcoreMesh(core_axis_name="c", subcore_axis_name="s")


@functools.partial(
    pl.kernel,
    mesh=_mesh,
    out_type=jax.ShapeDtypeStruct((NC, ROWS_ACC, D), jnp.float32),
    scratch_types=[
        pltpu.VMEM((KH, C), jnp.int32),
        pltpu.VMEM((KH, C), jnp.int32),
        pltpu.VMEM((C, D), jnp.float32),
        pltpu.VMEM((C, D), jnp.float32),
        pltpu.VMEM_SHARED((ROWS_ACC, D), jnp.float32),
        pltpu.SemaphoreType.DMA,
        pltpu.SemaphoreType.DMA,
    ],
)
def _sc_scatter(h_hbm, src_hbm, dst_hbm, zeros_hbm, out_hbm,
                src_v, dst_v, buf_a, buf_b, acc, sem_a, sem_b):
    c = lax.axis_index("c")
    s = lax.axis_index("s")
    w = s * NC + c
    pltpu.sync_copy(zeros_hbm, acc.at[pl.ds(s * RPS, RPS)])
    plsc.subcore_barrier()

    for p in range(2):
        pltpu.sync_copy(src_hbm.at[w, pl.ds(p * KH, KH)], src_v)
        pltpu.sync_copy(dst_hbm.at[w, pl.ds(p * KH, KH)], dst_v)
        pltpu.async_copy(h_hbm.at[src_v.at[0]], buf_a, sem_a)

        def body(i, carry):
            ja = 2 * i
            jb = 2 * i + 1
            pltpu.async_copy(h_hbm.at[src_v.at[jb]], buf_b, sem_b)
            pltpu.make_async_copy(h_hbm.at[src_v.at[ja]], buf_a, sem_a).wait()
            pltpu.sync_copy(buf_a, acc.at[dst_v.at[ja]], add=True)

            @pl.when(i < KH2 - 1)
            def _():
                pltpu.async_copy(h_hbm.at[src_v.at[ja + 2]], buf_a, sem_a)

            pltpu.make_async_copy(h_hbm.at[src_v.at[jb]], buf_b, sem_b).wait()
            pltpu.sync_copy(buf_b, acc.at[dst_v.at[jb]], add=True)
            return carry

        lax.fori_loop(0, KH2, body, 0, unroll=False)
    plsc.subcore_barrier()
    pltpu.sync_copy(acc.at[pl.ds(s * RPS, RPS)],
                    out_hbm.at[c, pl.ds(s * RPS, RPS)])


def _tc_mm_bn(p_ref, w_ref, g_ref, b_ref, o_ref):
    agg = p_ref[0, :N, :] + p_ref[1, :N, :]
    z = lax.dot_general(agg, w_ref[...], (((1,), (1,)), ((), ())),
                        preferred_element_type=jnp.float32)
    mean = jnp.mean(z, axis=0, keepdims=True)
    zc = z - mean
    var = jnp.mean(zc * zc, axis=0, keepdims=True)
    o_ref[...] = zc * lax.rsqrt(var + EPS) * g_ref[...] + b_ref[...]


def kernel(h, edge_index, W, gamma, beta):
    src = edge_index[0].astype(jnp.int32)
    dst = edge_index[1].astype(jnp.int32)
    pad = EPAD - E
    src = jnp.concatenate([src, jnp.zeros((pad,), jnp.int32)]).reshape(NW, K, C)
    dst = jnp.concatenate([dst, jnp.full((pad,), N, jnp.int32)]).reshape(NW, K, C)
    zeros = jnp.zeros((RPS, D), jnp.float32)

    partial = _sc_scatter(h, src, dst, zeros)

    out = pl.pallas_call(
        _tc_mm_bn,
        out_shape=jax.ShapeDtypeStruct((N, D), jnp.float32),
    )(partial, W, gamma.reshape(1, D), beta.reshape(1, D))
    return out


# speedup vs baseline: 4.0128x; 1.2463x over previous
import functools

import jax
import jax.numpy as jnp
from jax import lax
from jax.experimental import pallas as pl
from jax.experimental.pallas import tpu as pltpu
from jax.experimental.pallas import tpu_sc as plsc

N = 10000
D = 128
E = 320000
EPS = 1e-5

NC = 2
NS = 16
NW = NC * NS
C = 128
EPT = -(-E // NW)
K = 80
KH = K // 2
KH2 = KH // 2
EPAD = NW * K * C
ROWS_ACC = 10112
RPS = ROWS_ACC // NS

_mesh = plsc.VectorSubcoreMesh(core_axis_name="c", subcore_axis_name="s")


@functools.partial(
    pl.kernel,
    mesh=_mesh,
    out_type=jax.ShapeDtypeStruct((NC, ROWS_ACC, D), jnp.float32),
    scratch_types=[
        pltpu.VMEM((KH, C), jnp.int32),
        pltpu.VMEM((KH, C), jnp.int32),
        pltpu.VMEM((C, D), jnp.float32),
        pltpu.VMEM((C, D), jnp.float32),
        pltpu.VMEM_SHARED((ROWS_ACC, D), jnp.float32),
        pltpu.SemaphoreType.DMA,
        pltpu.SemaphoreType.DMA,
    ],
)
def _sc_scatter(h_hbm, src_hbm, dst_hbm, zeros_hbm, out_hbm,
                src_v, dst_v, buf_a, buf_b, acc, sem_a, sem_b):
    c = lax.axis_index("c")
    s = lax.axis_index("s")
    w = s * NC + c
    pltpu.sync_copy(zeros_hbm, acc.at[pl.ds(s * RPS, RPS)])
    plsc.subcore_barrier()

    for p in range(2):
        pltpu.sync_copy(src_hbm.at[w, pl.ds(p * KH, KH)], src_v)
        pltpu.sync_copy(dst_hbm.at[w, pl.ds(p * KH, KH)], dst_v)
        pltpu.async_copy(h_hbm.at[src_v.at[0]], buf_a, sem_a)

        def body(i, carry):
            ja = 2 * i
            jb = 2 * i + 1
            pltpu.async_copy(h_hbm.at[src_v.at[jb]], buf_b, sem_b)
            pltpu.make_async_copy(h_hbm.at[src_v.at[ja]], buf_a, sem_a).wait()
            pltpu.sync_copy(buf_a, acc.at[dst_v.at[ja]], add=True)

            @pl.when(i < KH2 - 1)
            def _():
                pltpu.async_copy(h_hbm.at[src_v.at[ja + 2]], buf_a, sem_a)

            pltpu.make_async_copy(h_hbm.at[src_v.at[jb]], buf_b, sem_b).wait()
            pltpu.sync_copy(buf_b, acc.at[dst_v.at[jb]], add=True)
            return carry

        lax.fori_loop(0, KH2, body, 0, unroll=False)
    plsc.subcore_barrier()
    pltpu.sync_copy(acc.at[pl.ds(s * RPS, RPS)],
                    out_hbm.at[c, pl.ds(s * RPS, RPS)])


def _tc_mm_bn(p_ref, w_ref, g_ref, b_ref, o_ref):
    agg = p_ref[0, :N, :] + p_ref[1, :N, :]
    z = lax.dot_general(agg, w_ref[...], (((1,), (1,)), ((), ())),
                        preferred_element_type=jnp.float32)
    mean = jnp.mean(z, axis=0, keepdims=True)
    zc = z - mean
    var = jnp.mean(zc * zc, axis=0, keepdims=True)
    o_ref[...] = zc * lax.rsqrt(var + EPS) * g_ref[...] + b_ref[...]


def kernel(h, edge_index, W, gamma, beta):
    src = edge_index[0].astype(jnp.int32).reshape(NW, EPT)
    dst = edge_index[1].astype(jnp.int32).reshape(NW, EPT)
    ppw = K * C - EPT
    w_ids = jnp.arange(NW, dtype=jnp.int32)[:, None]
    pad_i = jnp.arange(ppw, dtype=jnp.int32)[None, :]
    trash = N + (w_ids // NC) * 7 + pad_i % 7
    src = jnp.concatenate([src, jnp.zeros((NW, ppw), jnp.int32)], axis=1)
    dst = jnp.concatenate([dst, trash], axis=1)
    src = src.reshape(NW, K, C)
    dst = dst.reshape(NW, K, C)
    zeros = jnp.zeros((RPS, D), jnp.float32)

    partial = _sc_scatter(h, src, dst, zeros)

    out = pl.pallas_call(
        _tc_mm_bn,
        out_shape=jax.ShapeDtypeStruct((N, D), jnp.float32),
    )(partial, W, gamma.reshape(1, D), beta.reshape(1, D))
    return out


# speedup vs baseline: 11.2815x; 2.8114x over previous
import functools

import jax
import jax.numpy as jnp
from jax import lax
from jax.experimental import pallas as pl
from jax.experimental.pallas import tpu as pltpu
from jax.experimental.pallas import tpu_sc as plsc

N = 10000
D = 128
E = 320000
EPS = 1e-5

NC = 2
NS = 16
NW = NC * NS
C = 125
EPT = -(-E // NW)
K = 80
KH = K // 2
KH2 = KH // 2
EPAD = NW * K * C
ROWS_ACC = 10112
RPS = ROWS_ACC // NS

_mesh = plsc.VectorSubcoreMesh(core_axis_name="c", subcore_axis_name="s")


@functools.partial(
    pl.kernel,
    mesh=_mesh,
    out_type=jax.ShapeDtypeStruct((NC, ROWS_ACC, D), jnp.float32),
    scratch_types=[
        pltpu.VMEM((KH, C), jnp.int32),
        pltpu.VMEM((KH, C), jnp.int32),
        pltpu.VMEM((C, D), jnp.float32),
        pltpu.VMEM((C, D), jnp.float32),
        pltpu.VMEM_SHARED((ROWS_ACC, D), jnp.float32),
        pltpu.SemaphoreType.DMA,
        pltpu.SemaphoreType.DMA,
    ],
)
def _sc_scatter(h_hbm, src_hbm, dst_hbm, zeros_hbm, out_hbm,
                src_v, dst_v, buf_a, buf_b, acc, sem_a, sem_b):
    c = lax.axis_index("c")
    s = lax.axis_index("s")
    w = s * NC + c
    pltpu.sync_copy(zeros_hbm, acc.at[pl.ds(s * RPS, RPS)])
    plsc.subcore_barrier()

    for p in range(2):
        pltpu.sync_copy(src_hbm.at[w, pl.ds(p * KH, KH)], src_v)
        pltpu.sync_copy(dst_hbm.at[w, pl.ds(p * KH, KH)], dst_v)
        pltpu.async_copy(h_hbm.at[src_v.at[0]], buf_a, sem_a)

        def body(i, carry):
            ja = 2 * i
            jb = 2 * i + 1
            pltpu.async_copy(h_hbm.at[src_v.at[jb]], buf_b, sem_b)
            pltpu.make_async_copy(h_hbm.at[src_v.at[ja]], buf_a, sem_a).wait()
            pltpu.sync_copy(buf_a, acc.at[dst_v.at[ja]], add=True)

            @pl.when(i < KH2 - 1)
            def _():
                pltpu.async_copy(h_hbm.at[src_v.at[ja + 2]], buf_a, sem_a)

            pltpu.make_async_copy(h_hbm.at[src_v.at[jb]], buf_b, sem_b).wait()
            pltpu.sync_copy(buf_b, acc.at[dst_v.at[jb]], add=True)
            return carry

        lax.fori_loop(0, KH2, body, 0, unroll=False)
    plsc.subcore_barrier()
    pltpu.sync_copy(acc.at[pl.ds(s * RPS, RPS)],
                    out_hbm.at[c, pl.ds(s * RPS, RPS)])


def _tc_mm_bn(p_ref, w_ref, g_ref, b_ref, o_ref):
    agg = p_ref[0, :N, :] + p_ref[1, :N, :]
    z = lax.dot_general(agg, w_ref[...], (((1,), (1,)), ((), ())),
                        preferred_element_type=jnp.float32)
    mean = jnp.mean(z, axis=0, keepdims=True)
    zc = z - mean
    var = jnp.mean(zc * zc, axis=0, keepdims=True)
    o_ref[...] = zc * lax.rsqrt(var + EPS) * g_ref[...] + b_ref[...]


def kernel(h, edge_index, W, gamma, beta):
    src = edge_index[0].astype(jnp.int32).reshape(NW, K, C)
    dst = edge_index[1].astype(jnp.int32).reshape(NW, K, C)
    zeros = jnp.zeros((RPS, D), jnp.float32)

    partial = _sc_scatter(h, src, dst, zeros)

    out = pl.pallas_call(
        _tc_mm_bn,
        out_shape=jax.ShapeDtypeStruct((N, D), jnp.float32),
    )(partial, W, gamma.reshape(1, D), beta.reshape(1, D))
    return out
